# Initial kernel scaffold; baseline (speedup 1.0000x reference)
import jax
import jax.numpy as jnp
from jax.experimental import pallas as pl


def kernel(x, edge_index, W, b):
    raise NotImplementedError("write your pallas kernel here")


# speedup vs baseline: 8.8345x; 8.8345x over previous
import functools
import jax
import jax.numpy as jnp
from jax import lax
from jax.experimental import pallas as pl
from jax.experimental.pallas import tpu as pltpu
from jax.experimental.pallas import tpu_sc as plsc

ALPHA = 0.1
KSTEPS = 10

NC = 2
NS = 16
CH = 128


def _pre_body(x_ref, w_ref, b_ref, o_ref):
    h = jnp.dot(x_ref[...], w_ref[...], preferred_element_type=jnp.float32)
    o_ref[...] = jnp.maximum(h + b_ref[...], 0.0)


def _prep_body(d_ref, invb_ref, dinv_ref, sq_ref):
    nb = d_ref.shape[0] // 2
    deg = d_ref[:nb] + d_ref[nb:] + 1.0
    dinv = lax.rsqrt(deg)
    invb_ref[...] = (1.0 - ALPHA) / deg
    dinv_ref[...] = dinv
    sq_ref[...] = deg * dinv


def _scale_body(h_ref, dinv_ref, u_ref, u0a_ref):
    u = h_ref[...] * dinv_ref[...]
    u_ref[...] = u
    u0a_ref[...] = ALPHA * u


def _update_body(p0_ref, p1_ref, u_ref, u0a_ref, invb_ref, o_ref):
    s = p0_ref[...] + p1_ref[...] + u_ref[...]
    o_ref[...] = invb_ref[...] * s + u0a_ref[...]


def _post_body(u_ref, sq_ref, o_ref):
    z = u_ref[...] * sq_ref[...]
    m = jnp.max(z, axis=1, keepdims=True)
    e = jnp.exp(z - m)
    s = jnp.sum(e, axis=1, keepdims=True)
    o_ref[...] = (z - m) - jnp.log(s)


def _make_deg_kernel(n_pad, cpt):
    mesh = plsc.VectorSubcoreMesh(core_axis_name="c", subcore_axis_name="s", num_cores=NC, num_subcores=NS)
    rows_per_tile = n_pad // NS

    @functools.partial(
        pl.kernel,
        out_type=jax.ShapeDtypeStruct((NC, n_pad), jnp.float32),
        mesh=mesh,
        scratch_types=[
            pltpu.VMEM((cpt, CH), jnp.int32),
            pltpu.VMEM((CH,), jnp.float32),
            pltpu.VMEM_SHARED((n_pad,), jnp.float32),
        ],
    )
    def deg_kernel(dst_hbm, ones_hbm, zeros_hbm, out_hbm, didx, ones_v, acc):
        c = lax.axis_index("c")
        s = lax.axis_index("s")
        wid = c * NS + s
        pltpu.sync_copy(dst_hbm.at[pl.ds(wid * cpt, cpt)], didx)
        pltpu.sync_copy(ones_hbm, ones_v)
        r0 = s * rows_per_tile
        pltpu.sync_copy(zeros_hbm.at[pl.ds(r0, rows_per_tile)],
                        acc.at[pl.ds(r0, rows_per_tile)])
        plsc.subcore_barrier()

        def chunk(j, carry):
            pltpu.sync_copy(ones_v, acc.at[didx.at[j]], add=True)
            return carry

        lax.fori_loop(0, cpt, chunk, 0)
        plsc.subcore_barrier()
        pltpu.sync_copy(acc.at[pl.ds(r0, rows_per_tile)],
                        out_hbm.at[c, pl.ds(r0, rows_per_tile)])

    return deg_kernel


def _make_spmm_kernel(n_pad, c_dim, cpt):
    mesh = plsc.VectorSubcoreMesh(core_axis_name="c", subcore_axis_name="s", num_cores=NC, num_subcores=NS)
    rows_per_tile = n_pad // NS

    @functools.partial(
        pl.kernel,
        out_type=jax.ShapeDtypeStruct((NC, n_pad, c_dim), jnp.float32),
        mesh=mesh,
        scratch_types=[
            pltpu.VMEM((cpt, CH), jnp.int32),
            pltpu.VMEM((cpt, CH), jnp.int32),
            pltpu.VMEM((CH, c_dim), jnp.float32),
            pltpu.SemaphoreType.DMA,
            pltpu.VMEM_SHARED((n_pad, c_dim), jnp.float32),
        ],
        compiler_params=pltpu.CompilerParams(use_tc_tiling_on_sc=False),
    )
    def spmm_kernel(u_hbm, src_hbm, dst_hbm, zeros_hbm, out_hbm,
                    sidx, didx, rowbuf, sem, agg):
        c = lax.axis_index("c")
        s = lax.axis_index("s")
        wid = c * NS + s
        pltpu.sync_copy(src_hbm.at[pl.ds(wid * cpt, cpt)], sidx)
        pltpu.sync_copy(dst_hbm.at[pl.ds(wid * cpt, cpt)], didx)
        r0 = s * rows_per_tile
        pltpu.sync_copy(zeros_hbm.at[pl.ds(r0, rows_per_tile)],
                        agg.at[pl.ds(r0, rows_per_tile)])
        plsc.subcore_barrier()

        def chunk(j, carry):
            pltpu.async_copy(u_hbm.at[sidx.at[j]], rowbuf, sem).wait()
            pltpu.sync_copy(rowbuf, agg.at[didx.at[j]], add=True)
            return carry

        lax.fori_loop(0, cpt, chunk, 0)
        plsc.subcore_barrier()
        pltpu.sync_copy(agg.at[pl.ds(r0, rows_per_tile)],
                        out_hbm.at[c, pl.ds(r0, rows_per_tile)])

    return spmm_kernel


def kernel(x, edge_index, W, b):
    n, d = x.shape
    c_dim = W.shape[1]
    e = edge_index.shape[1]

    blk = 1024
    n_pad = ((n + blk - 1) // blk) * blk
    ept = -(-e // (NC * NS))
    cpt = -(-ept // CH)
    cpt = ((cpt + 7) // 8) * 8
    e_pad = NC * NS * cpt * CH

    src = edge_index[0]
    dst = edge_index[1]
    pad_e = e_pad - e
    src_p = jnp.concatenate(
        [src, jnp.zeros((pad_e,), dtype=src.dtype)]).reshape(-1, CH)
    dst_p = jnp.concatenate(
        [dst, jnp.full((pad_e,), n_pad - 1, dtype=dst.dtype)]).reshape(-1, CH)

    x_p = jnp.zeros((n_pad, d), x.dtype).at[:n].set(x)
    zeros2d = jnp.zeros((n_pad, c_dim), jnp.float32)
    zeros1d = jnp.zeros((n_pad,), jnp.float32)
    ones_ch = jnp.ones((CH,), jnp.float32)

    grid = n_pad // blk

    h0 = pl.pallas_call(
        _pre_body,
        grid=(grid,),
        in_specs=[
            pl.BlockSpec((blk, d), lambda i: (i, 0)),
            pl.BlockSpec((d, c_dim), lambda i: (0, 0)),
            pl.BlockSpec((1, c_dim), lambda i: (0, 0)),
        ],
        out_specs=pl.BlockSpec((blk, c_dim), lambda i: (i, 0)),
        out_shape=jax.ShapeDtypeStruct((n_pad, c_dim), jnp.float32),
    )(x_p, W, b.reshape(1, c_dim))

    deg_kernel = _make_deg_kernel(n_pad, cpt)
    degs = deg_kernel(dst_p, ones_ch, zeros1d)

    nb = n_pad // 128
    degs2 = degs.reshape(2 * nb, 128)
    invb80, dinv80, sq80 = pl.pallas_call(
        _prep_body,
        out_shape=[jax.ShapeDtypeStruct((nb, 128), jnp.float32)] * 3,
    )(degs2)

    dinv_b = jnp.broadcast_to(dinv80.reshape(n_pad, 1), (n_pad, c_dim))
    invb_b = jnp.broadcast_to(invb80.reshape(n_pad, 1), (n_pad, c_dim))
    sq_b = jnp.broadcast_to(sq80.reshape(n_pad, 1), (n_pad, c_dim))

    u, u0a = pl.pallas_call(
        _scale_body,
        grid=(grid,),
        in_specs=[pl.BlockSpec((blk, c_dim), lambda i: (i, 0))] * 2,
        out_specs=[pl.BlockSpec((blk, c_dim), lambda i: (i, 0))] * 2,
        out_shape=[jax.ShapeDtypeStruct((n_pad, c_dim), jnp.float32)] * 2,
    )(h0, dinv_b)

    spmm_kernel = _make_spmm_kernel(n_pad, c_dim, cpt)

    update = pl.pallas_call(
        _update_body,
        grid=(grid,),
        in_specs=[pl.BlockSpec((blk, c_dim), lambda i: (i, 0))] * 5,
        out_specs=pl.BlockSpec((blk, c_dim), lambda i: (i, 0)),
        out_shape=jax.ShapeDtypeStruct((n_pad, c_dim), jnp.float32),
    )

    for _ in range(KSTEPS):
        p = spmm_kernel(u, src_p, dst_p, zeros2d)
        u = update(p[0], p[1], u, u0a, invb_b)

    out = pl.pallas_call(
        _post_body,
        grid=(grid,),
        in_specs=[pl.BlockSpec((blk, c_dim), lambda i: (i, 0))] * 2,
        out_specs=pl.BlockSpec((blk, c_dim), lambda i: (i, 0)),
        out_shape=jax.ShapeDtypeStruct((n_pad, c_dim), jnp.float32),
    )(u, sq_b)

    return out[:n]


# speedup vs baseline: 10.2852x; 1.1642x over previous
import functools
import jax
import jax.numpy as jnp
from jax import lax
from jax.experimental import pallas as pl
from jax.experimental.pallas import tpu as pltpu
from jax.experimental.pallas import tpu_sc as plsc

ALPHA = 0.1
KSTEPS = 10

NC = 2
NS = 16
CH = 128
NB = 8


def _pre_body(x_ref, w_ref, b_ref, o_ref):
    h = jnp.dot(x_ref[...], w_ref[...], preferred_element_type=jnp.float32)
    o_ref[...] = jnp.maximum(h + b_ref[...], 0.0)


def _prep_body(d_ref, invb_ref, dinv_ref, sq_ref):
    nb = d_ref.shape[0] // 2
    deg = d_ref[:nb] + d_ref[nb:] + 1.0
    dinv = lax.rsqrt(deg)
    invb_ref[...] = (1.0 - ALPHA) / deg
    dinv_ref[...] = dinv
    sq_ref[...] = deg * dinv


def _scale_body(h_ref, dinv_ref, u_ref, u0a_ref):
    u = h_ref[...] * dinv_ref[...]
    u_ref[...] = u
    u0a_ref[...] = ALPHA * u


def _update_body(p0_ref, p1_ref, u_ref, u0a_ref, invb_ref, o_ref):
    s = p0_ref[...] + p1_ref[...] + u_ref[...]
    o_ref[...] = invb_ref[...] * s + u0a_ref[...]


def _post_body(u_ref, sq_ref, o_ref):
    z = u_ref[...] * sq_ref[...]
    m = jnp.max(z, axis=1, keepdims=True)
    e = jnp.exp(z - m)
    s = jnp.sum(e, axis=1, keepdims=True)
    o_ref[...] = (z - m) - jnp.log(s)


def _make_deg_kernel(n_pad, cpt):
    mesh = plsc.VectorSubcoreMesh(core_axis_name="c", subcore_axis_name="s", num_cores=NC, num_subcores=NS)
    rows_per_tile = n_pad // NS

    @functools.partial(
        pl.kernel,
        out_type=jax.ShapeDtypeStruct((NC, n_pad), jnp.float32),
        mesh=mesh,
        scratch_types=[
            pltpu.VMEM((cpt, CH), jnp.int32),
            pltpu.VMEM((CH,), jnp.float32),
            pltpu.VMEM_SHARED((n_pad,), jnp.float32),
        ],
    )
    def deg_kernel(dst_hbm, ones_hbm, zeros_hbm, out_hbm, didx, ones_v, acc):
        c = lax.axis_index("c")
        s = lax.axis_index("s")
        wid = c * NS + s
        pltpu.sync_copy(dst_hbm.at[pl.ds(wid * cpt, cpt)], didx)
        pltpu.sync_copy(ones_hbm, ones_v)
        r0 = s * rows_per_tile
        pltpu.sync_copy(zeros_hbm.at[pl.ds(r0, rows_per_tile)],
                        acc.at[pl.ds(r0, rows_per_tile)])
        plsc.subcore_barrier()

        def chunk(j, carry):
            pltpu.sync_copy(ones_v, acc.at[didx.at[j]], add=True)
            return carry

        lax.fori_loop(0, cpt, chunk, 0)
        plsc.subcore_barrier()
        pltpu.sync_copy(acc.at[pl.ds(r0, rows_per_tile)],
                        out_hbm.at[c, pl.ds(r0, rows_per_tile)])

    return deg_kernel


def _make_spmm_kernel(n_pad, c_dim, cpt):
    mesh = plsc.VectorSubcoreMesh(core_axis_name="c", subcore_axis_name="s", num_cores=NC, num_subcores=NS)
    rows_per_tile = n_pad // NS

    @functools.partial(
        pl.kernel,
        out_type=jax.ShapeDtypeStruct((NC, n_pad, c_dim), jnp.float32),
        mesh=mesh,
        scratch_types=[
            pltpu.VMEM((cpt, CH), jnp.int32),
            pltpu.VMEM((cpt, CH), jnp.int32),
            pltpu.VMEM((NB, CH, c_dim), jnp.float32),
            pltpu.SemaphoreType.DMA,
            pltpu.SemaphoreType.DMA,
            pltpu.VMEM_SHARED((n_pad, c_dim), jnp.float32),
        ],
        compiler_params=pltpu.CompilerParams(use_tc_tiling_on_sc=False),
    )
    def spmm_kernel(u_hbm, src_hbm, dst_hbm, zeros_hbm, out_hbm,
                    sidx, didx, rowbuf, gsem, ssem, agg):
        c = lax.axis_index("c")
        s = lax.axis_index("s")
        wid = c * NS + s
        pltpu.sync_copy(src_hbm.at[pl.ds(wid * cpt, cpt)], sidx)
        pltpu.sync_copy(dst_hbm.at[pl.ds(wid * cpt, cpt)], didx)
        r0 = s * rows_per_tile
        pltpu.sync_copy(zeros_hbm.at[pl.ds(r0, rows_per_tile)],
                        agg.at[pl.ds(r0, rows_per_tile)])
        plsc.subcore_barrier()

        def group(g, carry):
            base = g * NB
            gd = [pltpu.async_copy(u_hbm.at[sidx.at[base + b]],
                                   rowbuf.at[b], gsem)
                  for b in range(NB)]
            sd = []
            for b in range(NB):
                gd[b].wait()
                sd.append(pltpu.async_copy(rowbuf.at[b],
                                           agg.at[didx.at[base + b]],
                                           ssem, add=True))
            for b in range(NB):
                sd[b].wait()
            return carry

        lax.fori_loop(0, cpt // NB, group, 0)
        plsc.subcore_barrier()
        pltpu.sync_copy(agg.at[pl.ds(r0, rows_per_tile)],
                        out_hbm.at[c, pl.ds(r0, rows_per_tile)])

    return spmm_kernel


def kernel(x, edge_index, W, b):
    n, d = x.shape
    c_dim = W.shape[1]
    e = edge_index.shape[1]

    blk = 1024
    n_pad = ((n + blk - 1) // blk) * blk
    ept = -(-e // (NC * NS))
    cpt = -(-ept // CH)
    q = max(8, NB)
    cpt = ((cpt + q - 1) // q) * q
    e_pad = NC * NS * cpt * CH

    src = edge_index[0]
    dst = edge_index[1]
    pad_e = e_pad - e
    src_p = jnp.concatenate(
        [src, jnp.zeros((pad_e,), dtype=src.dtype)]).reshape(-1, CH)
    dst_p = jnp.concatenate(
        [dst, jnp.full((pad_e,), n_pad - 1, dtype=dst.dtype)]).reshape(-1, CH)

    x_p = jnp.zeros((n_pad, d), x.dtype).at[:n].set(x)
    zeros2d = jnp.zeros((n_pad, c_dim), jnp.float32)
    zeros1d = jnp.zeros((n_pad,), jnp.float32)
    ones_ch = jnp.ones((CH,), jnp.float32)

    grid = n_pad // blk

    h0 = pl.pallas_call(
        _pre_body,
        grid=(grid,),
        in_specs=[
            pl.BlockSpec((blk, d), lambda i: (i, 0)),
            pl.BlockSpec((d, c_dim), lambda i: (0, 0)),
            pl.BlockSpec((1, c_dim), lambda i: (0, 0)),
        ],
        out_specs=pl.BlockSpec((blk, c_dim), lambda i: (i, 0)),
        out_shape=jax.ShapeDtypeStruct((n_pad, c_dim), jnp.float32),
    )(x_p, W, b.reshape(1, c_dim))

    deg_kernel = _make_deg_kernel(n_pad, cpt)
    degs = deg_kernel(dst_p, ones_ch, zeros1d)

    nb = n_pad // 128
    degs2 = degs.reshape(2 * nb, 128)
    invb80, dinv80, sq80 = pl.pallas_call(
        _prep_body,
        out_shape=[jax.ShapeDtypeStruct((nb, 128), jnp.float32)] * 3,
    )(degs2)

    dinv_b = jnp.broadcast_to(dinv80.reshape(n_pad, 1), (n_pad, c_dim))
    invb_b = jnp.broadcast_to(invb80.reshape(n_pad, 1), (n_pad, c_dim))
    sq_b = jnp.broadcast_to(sq80.reshape(n_pad, 1), (n_pad, c_dim))

    u, u0a = pl.pallas_call(
        _scale_body,
        grid=(grid,),
        in_specs=[pl.BlockSpec((blk, c_dim), lambda i: (i, 0))] * 2,
        out_specs=[pl.BlockSpec((blk, c_dim), lambda i: (i, 0))] * 2,
        out_shape=[jax.ShapeDtypeStruct((n_pad, c_dim), jnp.float32)] * 2,
    )(h0, dinv_b)

    spmm_kernel = _make_spmm_kernel(n_pad, c_dim, cpt)

    update = pl.pallas_call(
        _update_body,
        grid=(grid,),
        in_specs=[pl.BlockSpec((blk, c_dim), lambda i: (i, 0))] * 5,
        out_specs=pl.BlockSpec((blk, c_dim), lambda i: (i, 0)),
        out_shape=jax.ShapeDtypeStruct((n_pad, c_dim), jnp.float32),
    )

    for _ in range(KSTEPS):
        p = spmm_kernel(u, src_p, dst_p, zeros2d)
        u = update(p[0], p[1], u, u0a, invb_b)

    out = pl.pallas_call(
        _post_body,
        grid=(grid,),
        in_specs=[pl.BlockSpec((blk, c_dim), lambda i: (i, 0))] * 2,
        out_specs=pl.BlockSpec((blk, c_dim), lambda i: (i, 0)),
        out_shape=jax.ShapeDtypeStruct((n_pad, c_dim), jnp.float32),
    )(u, sq_b)

    return out[:n]


# speedup vs baseline: 18.5562x; 1.8042x over previous
import functools
import jax
import jax.numpy as jnp
from jax import lax
from jax.experimental import pallas as pl
from jax.experimental.pallas import tpu as pltpu
from jax.experimental.pallas import tpu_sc as plsc

ALPHA = 0.1
KSTEPS = 10

NC = 2
NS = 16
CH = 128
NB = 8


def _pre_body(x_ref, w_ref, b_ref, o_ref):
    h = jnp.dot(x_ref[...], w_ref[...], preferred_element_type=jnp.float32)
    o_ref[...] = jnp.maximum(h + b_ref[...], 0.0)


def _prep_body(d_ref, invb_ref, dinv_ref, sq_ref):
    nb = d_ref.shape[0] // 2
    deg = d_ref[:nb] + d_ref[nb:] + 1.0
    dinv = lax.rsqrt(deg)
    invb_ref[...] = (1.0 - ALPHA) / deg
    dinv_ref[...] = dinv
    sq_ref[...] = deg * dinv


def _scale_body(h_ref, dinv_ref, u_ref, u0a_ref):
    u = h_ref[...] * dinv_ref[...]
    u_ref[...] = u
    u0a_ref[...] = ALPHA * u


def _update_body(p0a_ref, p0b_ref, p1a_ref, p1b_ref, u_ref, u0a_ref,
                 invb_ref, o_ref):
    pa = p0a_ref[...] + p1a_ref[...]
    pb = p0b_ref[...] + p1b_ref[...]
    s = jnp.concatenate([pa, pb], axis=1) + u_ref[...]
    o_ref[...] = invb_ref[...] * s + u0a_ref[...]


def _post_body(u_ref, sq_ref, o_ref):
    z = u_ref[...] * sq_ref[...]
    m = jnp.max(z, axis=1, keepdims=True)
    e = jnp.exp(z - m)
    s = jnp.sum(e, axis=1, keepdims=True)
    o_ref[...] = (z - m) - jnp.log(s)


def _make_deg_kernel(n_pad, cpt):
    mesh = plsc.VectorSubcoreMesh(core_axis_name="c", subcore_axis_name="s", num_cores=NC, num_subcores=NS)
    rows_per_tile = n_pad // NS

    @functools.partial(
        pl.kernel,
        out_type=jax.ShapeDtypeStruct((NC, n_pad), jnp.float32),
        mesh=mesh,
        scratch_types=[
            pltpu.VMEM((cpt, CH), jnp.int32),
            pltpu.VMEM((CH,), jnp.float32),
            pltpu.VMEM_SHARED((n_pad,), jnp.float32),
        ],
    )
    def deg_kernel(dst_hbm, ones_hbm, zeros_hbm, out_hbm, didx, ones_v, acc):
        c = lax.axis_index("c")
        s = lax.axis_index("s")
        wid = c * NS + s
        pltpu.sync_copy(dst_hbm.at[pl.ds(wid * cpt, cpt)], didx)
        pltpu.sync_copy(ones_hbm, ones_v)
        r0 = s * rows_per_tile
        pltpu.sync_copy(zeros_hbm.at[pl.ds(r0, rows_per_tile)],
                        acc.at[pl.ds(r0, rows_per_tile)])
        plsc.subcore_barrier()

        def chunk(j, carry):
            pltpu.sync_copy(ones_v, acc.at[didx.at[j]], add=True)
            return carry

        lax.fori_loop(0, cpt, chunk, 0)
        plsc.subcore_barrier()
        pltpu.sync_copy(acc.at[pl.ds(r0, rows_per_tile)],
                        out_hbm.at[c, pl.ds(r0, rows_per_tile)])

    return deg_kernel


def _make_spmm_kernel(n_pad, c_dim, cpt):
    mesh = plsc.VectorSubcoreMesh(core_axis_name="c", subcore_axis_name="s", num_cores=NC, num_subcores=NS)
    rows_per_tile = n_pad // NS
    ch = c_dim // 2

    @functools.partial(
        pl.kernel,
        out_type=jax.ShapeDtypeStruct((NC, 2, n_pad, ch), jnp.float32),
        mesh=mesh,
        scratch_types=[
            pltpu.VMEM((cpt, CH), jnp.int32),
            pltpu.VMEM((cpt, CH), jnp.int32),
            pltpu.VMEM((NB, CH, ch), jnp.float32),
            pltpu.VMEM((n_pad // NS, ch), jnp.float32),
            pltpu.SemaphoreType.DMA,
            pltpu.SemaphoreType.DMA,
            pltpu.VMEM_SHARED((n_pad, ch), jnp.float32),
            pltpu.VMEM_SHARED((n_pad, ch), jnp.float32),
        ],
        compiler_params=pltpu.CompilerParams(use_tc_tiling_on_sc=False),
    )
    def spmm_kernel(u_hbm, src_hbm, dst_hbm, out_hbm,
                    sidx, didx, rowbuf, zbuf, gsem, ssem, agg, ush):
        c = lax.axis_index("c")
        s = lax.axis_index("s")
        wid = c * NS + s
        pltpu.sync_copy(src_hbm.at[pl.ds(wid * cpt, cpt)], sidx)
        pltpu.sync_copy(dst_hbm.at[pl.ds(wid * cpt, cpt)], didx)
        r0 = s * rows_per_tile
        z16 = jnp.zeros((16,), jnp.float32)

        def zrow(i, carry):
            for k in range(ch // 16):
                zbuf[i, pl.ds(k * 16, 16)] = z16
            return carry

        lax.fori_loop(0, rows_per_tile, zrow, 0)

        for half in range(2):
            pltpu.sync_copy(
                u_hbm.at[pl.ds(r0, rows_per_tile), pl.ds(half * ch, ch)],
                ush.at[pl.ds(r0, rows_per_tile)])
            pltpu.sync_copy(zbuf, agg.at[pl.ds(r0, rows_per_tile)])
            plsc.subcore_barrier()

            def group(g, carry):
                base = g * NB
                gd = [pltpu.async_copy(ush.at[sidx.at[base + b]],
                                       rowbuf.at[b], gsem)
                      for b in range(NB)]
                sd = []
                for b in range(NB):
                    gd[b].wait()
                    sd.append(pltpu.async_copy(rowbuf.at[b],
                                               agg.at[didx.at[base + b]],
                                               ssem, add=True))
                for b in range(NB):
                    sd[b].wait()
                return carry

            lax.fori_loop(0, cpt // NB, group, 0)
            plsc.subcore_barrier()
            pltpu.sync_copy(agg.at[pl.ds(r0, rows_per_tile)],
                            out_hbm.at[c, half, pl.ds(r0, rows_per_tile)])

    return spmm_kernel


def kernel(x, edge_index, W, b):
    n, d = x.shape
    c_dim = W.shape[1]
    e = edge_index.shape[1]

    blk = 1024
    n_pad = ((n + blk - 1) // blk) * blk
    ept = -(-e // (NC * NS))
    cpt = -(-ept // CH)
    q = max(8, NB)
    cpt = ((cpt + q - 1) // q) * q
    e_pad = NC * NS * cpt * CH

    src = edge_index[0]
    dst = edge_index[1]
    pad_e = e_pad - e
    src_p = jnp.concatenate(
        [src, jnp.zeros((pad_e,), dtype=src.dtype)]).reshape(-1, CH)
    dst_p = jnp.concatenate(
        [dst, jnp.full((pad_e,), n_pad - 1, dtype=dst.dtype)]).reshape(-1, CH)

    x_p = jnp.zeros((n_pad, d), x.dtype).at[:n].set(x)
    zeros1d = jnp.zeros((n_pad,), jnp.float32)
    ones_ch = jnp.ones((CH,), jnp.float32)

    grid = n_pad // blk

    h0 = pl.pallas_call(
        _pre_body,
        grid=(grid,),
        in_specs=[
            pl.BlockSpec((blk, d), lambda i: (i, 0)),
            pl.BlockSpec((d, c_dim), lambda i: (0, 0)),
            pl.BlockSpec((1, c_dim), lambda i: (0, 0)),
        ],
        out_specs=pl.BlockSpec((blk, c_dim), lambda i: (i, 0)),
        out_shape=jax.ShapeDtypeStruct((n_pad, c_dim), jnp.float32),
    )(x_p, W, b.reshape(1, c_dim))

    deg_kernel = _make_deg_kernel(n_pad, cpt)
    degs = deg_kernel(dst_p, ones_ch, zeros1d)

    nb = n_pad // 128
    degs2 = degs.reshape(2 * nb, 128)
    invb80, dinv80, sq80 = pl.pallas_call(
        _prep_body,
        out_shape=[jax.ShapeDtypeStruct((nb, 128), jnp.float32)] * 3,
    )(degs2)

    dinv_b = jnp.broadcast_to(dinv80.reshape(n_pad, 1), (n_pad, c_dim))
    invb_b = jnp.broadcast_to(invb80.reshape(n_pad, 1), (n_pad, c_dim))
    sq_b = jnp.broadcast_to(sq80.reshape(n_pad, 1), (n_pad, c_dim))

    u, u0a = pl.pallas_call(
        _scale_body,
        grid=(grid,),
        in_specs=[pl.BlockSpec((blk, c_dim), lambda i: (i, 0))] * 2,
        out_specs=[pl.BlockSpec((blk, c_dim), lambda i: (i, 0))] * 2,
        out_shape=[jax.ShapeDtypeStruct((n_pad, c_dim), jnp.float32)] * 2,
    )(h0, dinv_b)

    spmm_kernel = _make_spmm_kernel(n_pad, c_dim, cpt)

    chalf = c_dim // 2
    update = pl.pallas_call(
        _update_body,
        grid=(grid,),
        in_specs=[pl.BlockSpec((blk, chalf), lambda i: (i, 0))] * 4
        + [pl.BlockSpec((blk, c_dim), lambda i: (i, 0))] * 3,
        out_specs=pl.BlockSpec((blk, c_dim), lambda i: (i, 0)),
        out_shape=jax.ShapeDtypeStruct((n_pad, c_dim), jnp.float32),
    )

    for _ in range(KSTEPS):
        p = spmm_kernel(u, src_p, dst_p)
        u = update(p[0, 0], p[0, 1], p[1, 0], p[1, 1], u, u0a, invb_b)

    out = pl.pallas_call(
        _post_body,
        grid=(grid,),
        in_specs=[pl.BlockSpec((blk, c_dim), lambda i: (i, 0))] * 2,
        out_specs=pl.BlockSpec((blk, c_dim), lambda i: (i, 0)),
        out_shape=jax.ShapeDtypeStruct((n_pad, c_dim), jnp.float32),
    )(u, sq_b)

    return out[:n]


# speedup vs baseline: 19.2756x; 1.0388x over previous
import functools
import jax
import jax.numpy as jnp
from jax import lax
from jax.experimental import pallas as pl
from jax.experimental.pallas import tpu as pltpu
from jax.experimental.pallas import tpu_sc as plsc

ALPHA = 0.1
KSTEPS = 10

NC = 2
NS = 16
CH = 128
NB = 8


def _pre_body(x_ref, w_ref, b_ref, o_ref):
    h = jnp.dot(x_ref[...], w_ref[...], preferred_element_type=jnp.float32)
    o_ref[...] = jnp.maximum(h + b_ref[...], 0.0)


def _prep_body(d_ref, invb_ref, dinv_ref, sq_ref):
    nb = d_ref.shape[0] // 2
    deg = d_ref[:nb] + d_ref[nb:] + 1.0
    dinv = lax.rsqrt(deg)
    invb_ref[...] = (1.0 - ALPHA) / deg
    dinv_ref[...] = dinv
    sq_ref[...] = deg * dinv


def _scale_body(h_ref, dinv_ref, u_ref, u0a_ref):
    u = h_ref[...] * dinv_ref[...]
    u_ref[...] = u
    u0a_ref[...] = ALPHA * u


def _update_body(p0a_ref, p0b_ref, p1a_ref, p1b_ref, u_ref, u0a_ref,
                 invb_ref, o_ref):
    pa = p0a_ref[...] + p1a_ref[...]
    pb = p0b_ref[...] + p1b_ref[...]
    s = jnp.concatenate([pa, pb], axis=1) + u_ref[...]
    o_ref[...] = invb_ref[...] * s + u0a_ref[...]


def _post_body(u_ref, sq_ref, o_ref):
    z = u_ref[...] * sq_ref[...]
    m = jnp.max(z, axis=1, keepdims=True)
    e = jnp.exp(z - m)
    s = jnp.sum(e, axis=1, keepdims=True)
    o_ref[...] = (z - m) - jnp.log(s)


def _make_deg_kernel(n_pad, cpt):
    mesh = plsc.VectorSubcoreMesh(core_axis_name="c", subcore_axis_name="s", num_cores=NC, num_subcores=NS)
    rows_per_tile = n_pad // NS

    @functools.partial(
        pl.kernel,
        out_type=jax.ShapeDtypeStruct((NC, n_pad), jnp.float32),
        mesh=mesh,
        scratch_types=[
            pltpu.VMEM((cpt, CH), jnp.int32),
            pltpu.VMEM((CH,), jnp.float32),
            pltpu.VMEM_SHARED((n_pad,), jnp.float32),
        ],
    )
    def deg_kernel(dst_hbm, ones_hbm, zeros_hbm, out_hbm, didx, ones_v, acc):
        c = lax.axis_index("c")
        s = lax.axis_index("s")
        wid = c * NS + s
        pltpu.sync_copy(dst_hbm.at[pl.ds(wid * cpt, cpt)], didx)
        pltpu.sync_copy(ones_hbm, ones_v)
        r0 = s * rows_per_tile
        pltpu.sync_copy(zeros_hbm.at[pl.ds(r0, rows_per_tile)],
                        acc.at[pl.ds(r0, rows_per_tile)])
        plsc.subcore_barrier()

        def chunk(j, carry):
            pltpu.sync_copy(ones_v, acc.at[didx.at[j]], add=True)
            return carry

        lax.fori_loop(0, cpt, chunk, 0)
        plsc.subcore_barrier()
        pltpu.sync_copy(acc.at[pl.ds(r0, rows_per_tile)],
                        out_hbm.at[c, pl.ds(r0, rows_per_tile)])

    return deg_kernel


def _make_fused_kernel(n_pad, c_dim, cpt):
    mesh = plsc.VectorSubcoreMesh(core_axis_name="c", subcore_axis_name="s",
                                  num_cores=NC, num_subcores=NS)
    rpt = n_pad // NS
    urows = n_pad // (NC * NS)
    PH = 4
    ch = c_dim // PH
    zr = rpt // 4

    @functools.partial(
        pl.kernel,
        out_type=[
            jax.ShapeDtypeStruct((n_pad, c_dim), jnp.float32),
            jax.ShapeDtypeStruct((NC, 2, n_pad // 2, ch), jnp.float32),
        ],
        mesh=mesh,
        scratch_types=[
            pltpu.VMEM((cpt, CH), jnp.int32),
            pltpu.VMEM((cpt, CH), jnp.int32),
            pltpu.VMEM((NB, CH, ch), jnp.float32),
            pltpu.VMEM((zr, ch), jnp.float32),
            pltpu.VMEM((urows, ch), jnp.float32),
            pltpu.VMEM((urows, ch), jnp.float32),
            pltpu.VMEM((urows, c_dim), jnp.float32),
            pltpu.VMEM((urows, 16), jnp.float32),
            pltpu.VMEM((urows, c_dim), jnp.float32),
            pltpu.SemaphoreType.DMA,
            pltpu.SemaphoreType.DMA,
            pltpu.SemaphoreType.REGULAR,
            pltpu.VMEM_SHARED((n_pad, ch), jnp.float32),
            pltpu.VMEM_SHARED((n_pad, ch), jnp.float32),
        ],
        compiler_params=pltpu.CompilerParams(use_tc_tiling_on_sc=False),
    )
    def fused_kernel(u0a_hbm, invb16_hbm, src_hbm, dst_hbm,
                     u_hbm, p_hbm,
                     sidx, didx, rowbuf, zbuf, pbuf,
                     obuf, ubuf, invb16, u0ar, gsem, ssem, bsem, agg, ush):
        c = lax.axis_index("c")
        s = lax.axis_index("s")
        wid = c * NS + s
        r0 = s * rpt
        ur0 = wid * urows
        obase = (NC - 1 - c) * (n_pad // 2)

        pltpu.sync_copy(src_hbm.at[pl.ds(wid * cpt, cpt)], sidx)
        pltpu.sync_copy(dst_hbm.at[pl.ds(wid * cpt, cpt)], didx)
        pltpu.sync_copy(invb16_hbm.at[pl.ds(ur0, urows)], invb16)
        pltpu.sync_copy(u0a_hbm.at[pl.ds(ur0, urows)], u0ar)
        inv_alpha = jnp.full((16,), 1.0 / ALPHA, jnp.float32)

        def u0row(i, carry):
            for kq in range(c_dim // 16):
                ubuf[i, pl.ds(kq * 16, 16)] = (
                    u0ar[i, pl.ds(kq * 16, 16)] * inv_alpha)
            return carry

        lax.fori_loop(0, urows, u0row, 0)
        pltpu.sync_copy(ubuf, u_hbm.at[pl.ds(ur0, urows)])

        z16 = jnp.zeros((16,), jnp.float32)

        def zrow(i, carry):
            for k in range(ch // 16):
                zbuf[i, pl.ds(k * 16, 16)] = z16
            return carry

        lax.fori_loop(0, zr, zrow, 0)

        def xbarrier():
            plsc.subcore_barrier()

            @pl.when(s == 0)
            def _():
                pltpu.core_barrier(bsem, core_axis_name="c")

            plsc.subcore_barrier()

        xbarrier()

        def step(_k, carry):
            for half in range(PH):
                par = half % 2
                pltpu.sync_copy(
                    u_hbm.at[pl.ds(r0, rpt), pl.ds(half * ch, ch)],
                    ush.at[pl.ds(r0, rpt)])
                for q in range(4):
                    pltpu.sync_copy(zbuf, agg.at[pl.ds(r0 + q * zr, zr)])
                plsc.subcore_barrier()

                def group(g, carry2):
                    base = g * NB
                    gd = [pltpu.async_copy(ush.at[sidx.at[base + b]],
                                           rowbuf.at[b], gsem)
                          for b in range(NB)]
                    sd = []
                    for b in range(NB):
                        gd[b].wait()
                        sd.append(pltpu.async_copy(rowbuf.at[b],
                                                   agg.at[didx.at[base + b]],
                                                   ssem, add=True))
                    for b in range(NB):
                        sd[b].wait()
                    return carry2

                lax.fori_loop(0, cpt // NB, group, 0)
                plsc.subcore_barrier()
                pltpu.sync_copy(agg.at[pl.ds(ur0, urows)], pbuf)
                orow = obase + s * (n_pad // (2 * NS))
                pltpu.sync_copy(
                    agg.at[pl.ds(orow, n_pad // (2 * NS))],
                    p_hbm.at[c, par, pl.ds(s * (n_pad // (2 * NS)),
                                           n_pad // (2 * NS))])
                xbarrier()
                pltpu.sync_copy(
                    p_hbm.at[NC - 1 - c, par,
                             pl.ds(ur0 - c * (n_pad // 2), urows)],
                    obuf)

                def urow(i, carry3):
                    iv = invb16[i, pl.ds(0, 16)]
                    for kk in range(ch // 16):
                        col = half * ch + kk * 16
                        v = (pbuf[i, pl.ds(kk * 16, 16)]
                             + obuf[i, pl.ds(kk * 16, 16)]
                             + ubuf[i, pl.ds(col, 16)])
                        ubuf[i, pl.ds(col, 16)] = (
                            iv * v + u0ar[i, pl.ds(col, 16)])
                    return carry3

                lax.fori_loop(0, urows, urow, 0)
                pltpu.sync_copy(
                    ubuf.at[:, pl.ds(half * ch, ch)],
                    u_hbm.at[pl.ds(ur0, urows), pl.ds(half * ch, ch)])
            return carry

        lax.fori_loop(0, KSTEPS, step, 0)

    return fused_kernel


def _make_spmm_kernel(n_pad, c_dim, cpt):
    mesh = plsc.VectorSubcoreMesh(core_axis_name="c", subcore_axis_name="s", num_cores=NC, num_subcores=NS)
    rows_per_tile = n_pad // NS
    ch = c_dim // 2

    @functools.partial(
        pl.kernel,
        out_type=jax.ShapeDtypeStruct((NC, 2, n_pad, ch), jnp.float32),
        mesh=mesh,
        scratch_types=[
            pltpu.VMEM((cpt, CH), jnp.int32),
            pltpu.VMEM((cpt, CH), jnp.int32),
            pltpu.VMEM((NB, CH, ch), jnp.float32),
            pltpu.VMEM((n_pad // NS, ch), jnp.float32),
            pltpu.SemaphoreType.DMA,
            pltpu.SemaphoreType.DMA,
            pltpu.VMEM_SHARED((n_pad, ch), jnp.float32),
            pltpu.VMEM_SHARED((n_pad, ch), jnp.float32),
        ],
        compiler_params=pltpu.CompilerParams(use_tc_tiling_on_sc=False),
    )
    def spmm_kernel(u_hbm, src_hbm, dst_hbm, out_hbm,
                    sidx, didx, rowbuf, zbuf, gsem, ssem, agg, ush):
        c = lax.axis_index("c")
        s = lax.axis_index("s")
        wid = c * NS + s
        pltpu.sync_copy(src_hbm.at[pl.ds(wid * cpt, cpt)], sidx)
        pltpu.sync_copy(dst_hbm.at[pl.ds(wid * cpt, cpt)], didx)
        r0 = s * rows_per_tile
        z16 = jnp.zeros((16,), jnp.float32)

        def zrow(i, carry):
            for k in range(ch // 16):
                zbuf[i, pl.ds(k * 16, 16)] = z16
            return carry

        lax.fori_loop(0, rows_per_tile, zrow, 0)

        for half in range(2):
            pltpu.sync_copy(
                u_hbm.at[pl.ds(r0, rows_per_tile), pl.ds(half * ch, ch)],
                ush.at[pl.ds(r0, rows_per_tile)])
            pltpu.sync_copy(zbuf, agg.at[pl.ds(r0, rows_per_tile)])
            plsc.subcore_barrier()

            def group(g, carry):
                base = g * NB
                gd = [pltpu.async_copy(ush.at[sidx.at[base + b]],
                                       rowbuf.at[b], gsem)
                      for b in range(NB)]
                sd = []
                for b in range(NB):
                    gd[b].wait()
                    sd.append(pltpu.async_copy(rowbuf.at[b],
                                               agg.at[didx.at[base + b]],
                                               ssem, add=True))
                for b in range(NB):
                    sd[b].wait()
                return carry

            lax.fori_loop(0, cpt // NB, group, 0)
            plsc.subcore_barrier()
            pltpu.sync_copy(agg.at[pl.ds(r0, rows_per_tile)],
                            out_hbm.at[c, half, pl.ds(r0, rows_per_tile)])

    return spmm_kernel


def kernel(x, edge_index, W, b):
    n, d = x.shape
    c_dim = W.shape[1]
    e = edge_index.shape[1]

    blk = 1024
    n_pad = ((n + blk - 1) // blk) * blk
    ept = -(-e // (NC * NS))
    cpt = -(-ept // CH)
    q = max(8, NB)
    cpt = ((cpt + q - 1) // q) * q
    e_pad = NC * NS * cpt * CH

    src = edge_index[0]
    dst = edge_index[1]
    pad_e = e_pad - e
    src_p = jnp.concatenate(
        [src, jnp.zeros((pad_e,), dtype=src.dtype)]).reshape(-1, CH)
    dst_p = jnp.concatenate(
        [dst, jnp.full((pad_e,), n_pad - 1, dtype=dst.dtype)]).reshape(-1, CH)

    x_p = jnp.zeros((n_pad, d), x.dtype).at[:n].set(x)
    zeros1d = jnp.zeros((n_pad,), jnp.float32)
    ones_ch = jnp.ones((CH,), jnp.float32)

    grid = n_pad // blk

    h0 = pl.pallas_call(
        _pre_body,
        grid=(grid,),
        in_specs=[
            pl.BlockSpec((blk, d), lambda i: (i, 0)),
            pl.BlockSpec((d, c_dim), lambda i: (0, 0)),
            pl.BlockSpec((1, c_dim), lambda i: (0, 0)),
        ],
        out_specs=pl.BlockSpec((blk, c_dim), lambda i: (i, 0)),
        out_shape=jax.ShapeDtypeStruct((n_pad, c_dim), jnp.float32),
    )(x_p, W, b.reshape(1, c_dim))

    deg_kernel = _make_deg_kernel(n_pad, cpt)
    degs = deg_kernel(dst_p, ones_ch, zeros1d)

    nb = n_pad // 128
    degs2 = degs.reshape(2 * nb, 128)
    invb80, dinv80, sq80 = pl.pallas_call(
        _prep_body,
        out_shape=[jax.ShapeDtypeStruct((nb, 128), jnp.float32)] * 3,
    )(degs2)

    dinv_b = jnp.broadcast_to(dinv80.reshape(n_pad, 1), (n_pad, c_dim))
    invb16 = jnp.broadcast_to(invb80.reshape(n_pad, 1), (n_pad, 16))
    sq_b = jnp.broadcast_to(sq80.reshape(n_pad, 1), (n_pad, c_dim))

    u, u0a = pl.pallas_call(
        _scale_body,
        grid=(grid,),
        in_specs=[pl.BlockSpec((blk, c_dim), lambda i: (i, 0))] * 2,
        out_specs=[pl.BlockSpec((blk, c_dim), lambda i: (i, 0))] * 2,
        out_shape=[jax.ShapeDtypeStruct((n_pad, c_dim), jnp.float32)] * 2,
    )(h0, dinv_b)

    fused_kernel = _make_fused_kernel(n_pad, c_dim, cpt)
    u, _ = fused_kernel(u0a, invb16, src_p, dst_p)

    out = pl.pallas_call(
        _post_body,
        grid=(grid,),
        in_specs=[pl.BlockSpec((blk, c_dim), lambda i: (i, 0))] * 2,
        out_specs=pl.BlockSpec((blk, c_dim), lambda i: (i, 0)),
        out_shape=jax.ShapeDtypeStruct((n_pad, c_dim), jnp.float32),
    )(u, sq_b)

    return out[:n]


# speedup vs baseline: 27.8094x; 1.4427x over previous
import functools
import jax
import jax.numpy as jnp
from jax import lax
from jax.experimental import pallas as pl
from jax.experimental.pallas import tpu as pltpu
from jax.experimental.pallas import tpu_sc as plsc

ALPHA = 0.1
KSTEPS = 10

NC = 2
NS = 16
CH = 128
NB = 4


def _pre_body(x_ref, w_ref, b_ref, o_ref):
    h = jnp.dot(x_ref[...], w_ref[...], preferred_element_type=jnp.float32)
    o_ref[...] = jnp.maximum(h + b_ref[...], 0.0)


def _prep_body(d_ref, invb_ref, dinv_ref, sq_ref):
    nb = d_ref.shape[0] // 2
    deg = d_ref[:nb] + d_ref[nb:] + 1.0
    dinv = lax.rsqrt(deg)
    invb_ref[...] = (1.0 - ALPHA) / deg
    dinv_ref[...] = dinv
    sq_ref[...] = deg * dinv


def _scale_body(h_ref, dinv_ref, u_ref, u0a_ref):
    u = h_ref[...] * dinv_ref[...]
    u_ref[...] = u
    u0a_ref[...] = ALPHA * u


def _update_body(p0a_ref, p0b_ref, p1a_ref, p1b_ref, u_ref, u0a_ref,
                 invb_ref, o_ref):
    pa = p0a_ref[...] + p1a_ref[...]
    pb = p0b_ref[...] + p1b_ref[...]
    s = jnp.concatenate([pa, pb], axis=1) + u_ref[...]
    o_ref[...] = invb_ref[...] * s + u0a_ref[...]


def _post_body(u_ref, sq_ref, o_ref):
    z = u_ref[...] * sq_ref[...]
    m = jnp.max(z, axis=1, keepdims=True)
    e = jnp.exp(z - m)
    s = jnp.sum(e, axis=1, keepdims=True)
    o_ref[...] = (z - m) - jnp.log(s)


def _make_deg_kernel(n_pad, cpt):
    mesh = plsc.VectorSubcoreMesh(core_axis_name="c", subcore_axis_name="s", num_cores=NC, num_subcores=NS)
    rows_per_tile = n_pad // NS

    @functools.partial(
        pl.kernel,
        out_type=jax.ShapeDtypeStruct((NC, n_pad), jnp.float32),
        mesh=mesh,
        scratch_types=[
            pltpu.VMEM((cpt, CH), jnp.int32),
            pltpu.VMEM((CH,), jnp.float32),
            pltpu.VMEM_SHARED((n_pad,), jnp.float32),
        ],
    )
    def deg_kernel(dst_hbm, ones_hbm, zeros_hbm, out_hbm, didx, ones_v, acc):
        c = lax.axis_index("c")
        s = lax.axis_index("s")
        wid = c * NS + s
        pltpu.sync_copy(dst_hbm.at[pl.ds(wid * cpt, cpt)], didx)
        pltpu.sync_copy(ones_hbm, ones_v)
        r0 = s * rows_per_tile
        pltpu.sync_copy(zeros_hbm.at[pl.ds(r0, rows_per_tile)],
                        acc.at[pl.ds(r0, rows_per_tile)])
        plsc.subcore_barrier()

        def chunk(j, carry):
            pltpu.sync_copy(ones_v, acc.at[didx.at[j]], add=True)
            return carry

        lax.fori_loop(0, cpt, chunk, 0)
        plsc.subcore_barrier()
        pltpu.sync_copy(acc.at[pl.ds(r0, rows_per_tile)],
                        out_hbm.at[c, pl.ds(r0, rows_per_tile)])

    return deg_kernel


def _make_fused_kernel(n_pad, c_dim, cpt):
    mesh = plsc.VectorSubcoreMesh(core_axis_name="c", subcore_axis_name="s",
                                  num_cores=NC, num_subcores=NS)
    rpt = n_pad // NS
    ch = c_dim // NC
    zr = rpt // 4

    @functools.partial(
        pl.kernel,
        out_type=jax.ShapeDtypeStruct((NC, n_pad, ch), jnp.float32),
        mesh=mesh,
        scratch_types=[
            pltpu.VMEM((2, NB, CH), jnp.int32),
            pltpu.VMEM((2, NB, CH), jnp.int32),
            pltpu.VMEM((NB, CH, ch), jnp.float32),
            pltpu.VMEM((zr, ch), jnp.float32),
            pltpu.VMEM((rpt // 2, ch), jnp.float32),
            pltpu.VMEM((rpt, ch), jnp.float32),
            pltpu.VMEM((rpt, 16), jnp.float32),
            pltpu.VMEM((rpt, ch), jnp.float32),
            pltpu.SemaphoreType.DMA,
            pltpu.SemaphoreType.DMA,
            pltpu.SemaphoreType.DMA,
            pltpu.SemaphoreType.DMA,
            pltpu.VMEM_SHARED((n_pad, ch), jnp.float32),
            pltpu.VMEM_SHARED((n_pad, ch), jnp.float32),
        ],
        compiler_params=pltpu.CompilerParams(use_tc_tiling_on_sc=False),
    )
    def fused_kernel(u0a_hbm, invb16_hbm, src_hbm, dst_hbm,
                     u_out,
                     sidx, didx, rowbuf, zbuf, pbuf, ubuf, invb16,
                     u0ar, gsem, ssem, isem, jsem, agg, ush):
        c = lax.axis_index("c")
        s = lax.axis_index("s")
        r0 = s * rpt
        ngroups = cpt * 2 // NB
        gbase = s * (cpt * 2)

        pltpu.sync_copy(invb16_hbm.at[pl.ds(r0, rpt)], invb16)
        pltpu.sync_copy(
            u0a_hbm.at[pl.ds(r0, rpt), pl.ds(c * ch, ch)], u0ar)
        inv_alpha = jnp.full((16,), 1.0 / ALPHA, jnp.float32)

        def u0row(i, carry):
            for kq in range(ch // 16):
                ubuf[i, pl.ds(kq * 16, 16)] = (
                    u0ar[i, pl.ds(kq * 16, 16)] * inv_alpha)
            return carry

        lax.fori_loop(0, rpt, u0row, 0)
        pltpu.sync_copy(ubuf, ush.at[pl.ds(r0, rpt)])

        z16 = jnp.zeros((16,), jnp.float32)

        def zrow(i, carry):
            for k in range(ch // 16):
                zbuf[i, pl.ds(k * 16, 16)] = z16
            return carry

        lax.fori_loop(0, zr, zrow, 0)
        for q in range(4):
            pltpu.sync_copy(zbuf, agg.at[pl.ds(r0 + q * zr, zr)])
        plsc.subcore_barrier()

        def step(_k, carry):
            pltpu.async_copy(src_hbm.at[pl.ds(gbase, NB)], sidx.at[0], isem)
            pltpu.async_copy(dst_hbm.at[pl.ds(gbase, NB)], didx.at[0], isem)

            def process(slot):
                gd = [pltpu.async_copy(ush.at[sidx.at[slot, b]],
                                       rowbuf.at[b], gsem)
                      for b in range(NB)]
                sd = []
                for b in range(NB):
                    gd[b].wait()
                    sd.append(pltpu.async_copy(rowbuf.at[b],
                                               agg.at[didx.at[slot, b]],
                                               ssem, add=True))
                for b in range(NB):
                    sd[b].wait()

            def drain(slot, sem):
                pltpu.make_async_copy(src_hbm.at[pl.ds(gbase, NB)],
                                      sidx.at[slot], sem).wait()
                pltpu.make_async_copy(dst_hbm.at[pl.ds(gbase, NB)],
                                      didx.at[slot], sem).wait()

            def pair(h, carry2):
                g1 = h * 2 + 1
                pltpu.async_copy(src_hbm.at[pl.ds(gbase + g1 * NB, NB)],
                                 sidx.at[1], jsem)
                pltpu.async_copy(dst_hbm.at[pl.ds(gbase + g1 * NB, NB)],
                                 didx.at[1], jsem)
                drain(0, isem)
                process(0)

                @pl.when(h + 1 < ngroups // 2)
                def _():
                    g2 = h * 2 + 2
                    pltpu.async_copy(src_hbm.at[pl.ds(gbase + g2 * NB, NB)],
                                     sidx.at[0], isem)
                    pltpu.async_copy(dst_hbm.at[pl.ds(gbase + g2 * NB, NB)],
                                     didx.at[0], isem)

                drain(1, jsem)
                process(1)
                return carry2

            lax.fori_loop(0, ngroups // 2, pair, 0)
            plsc.subcore_barrier()
            hrpt = rpt // 2
            for hp in range(2):
                pltpu.sync_copy(agg.at[pl.ds(r0 + hp * hrpt, hrpt)], pbuf)
                for q in range(2):
                    pltpu.sync_copy(
                        zbuf, agg.at[pl.ds(r0 + hp * hrpt + q * zr, zr)])

                def urow(i, carry3):
                    ii = hp * hrpt + i
                    iv = invb16[ii, pl.ds(0, 16)]
                    for kk in range(ch // 16):
                        v = (pbuf[i, pl.ds(kk * 16, 16)]
                             + ubuf[ii, pl.ds(kk * 16, 16)])
                        ubuf[ii, pl.ds(kk * 16, 16)] = (
                            iv * v + u0ar[ii, pl.ds(kk * 16, 16)])
                    return carry3

                lax.fori_loop(0, hrpt, urow, 0)
            pltpu.sync_copy(ubuf, ush.at[pl.ds(r0, rpt)])
            plsc.subcore_barrier()
            return carry

        lax.fori_loop(0, KSTEPS, step, 0)
        pltpu.sync_copy(ubuf, u_out.at[c, pl.ds(r0, rpt)])

    return fused_kernel


def _make_spmm_kernel(n_pad, c_dim, cpt):
    mesh = plsc.VectorSubcoreMesh(core_axis_name="c", subcore_axis_name="s", num_cores=NC, num_subcores=NS)
    rows_per_tile = n_pad // NS
    ch = c_dim // 2

    @functools.partial(
        pl.kernel,
        out_type=jax.ShapeDtypeStruct((NC, 2, n_pad, ch), jnp.float32),
        mesh=mesh,
        scratch_types=[
            pltpu.VMEM((cpt, CH), jnp.int32),
            pltpu.VMEM((cpt, CH), jnp.int32),
            pltpu.VMEM((NB, CH, ch), jnp.float32),
            pltpu.VMEM((n_pad // NS, ch), jnp.float32),
            pltpu.SemaphoreType.DMA,
            pltpu.SemaphoreType.DMA,
            pltpu.VMEM_SHARED((n_pad, ch), jnp.float32),
            pltpu.VMEM_SHARED((n_pad, ch), jnp.float32),
        ],
        compiler_params=pltpu.CompilerParams(use_tc_tiling_on_sc=False),
    )
    def spmm_kernel(u_hbm, src_hbm, dst_hbm, out_hbm,
                    sidx, didx, rowbuf, zbuf, gsem, ssem, agg, ush):
        c = lax.axis_index("c")
        s = lax.axis_index("s")
        wid = c * NS + s
        pltpu.sync_copy(src_hbm.at[pl.ds(wid * cpt, cpt)], sidx)
        pltpu.sync_copy(dst_hbm.at[pl.ds(wid * cpt, cpt)], didx)
        r0 = s * rows_per_tile
        z16 = jnp.zeros((16,), jnp.float32)

        def zrow(i, carry):
            for k in range(ch // 16):
                zbuf[i, pl.ds(k * 16, 16)] = z16
            return carry

        lax.fori_loop(0, rows_per_tile, zrow, 0)

        for half in range(2):
            pltpu.sync_copy(
                u_hbm.at[pl.ds(r0, rows_per_tile), pl.ds(half * ch, ch)],
                ush.at[pl.ds(r0, rows_per_tile)])
            pltpu.sync_copy(zbuf, agg.at[pl.ds(r0, rows_per_tile)])
            plsc.subcore_barrier()

            def group(g, carry):
                base = g * NB
                gd = [pltpu.async_copy(ush.at[sidx.at[base + b]],
                                       rowbuf.at[b], gsem)
                      for b in range(NB)]
                sd = []
                for b in range(NB):
                    gd[b].wait()
                    sd.append(pltpu.async_copy(rowbuf.at[b],
                                               agg.at[didx.at[base + b]],
                                               ssem, add=True))
                for b in range(NB):
                    sd[b].wait()
                return carry

            lax.fori_loop(0, cpt // NB, group, 0)
            plsc.subcore_barrier()
            pltpu.sync_copy(agg.at[pl.ds(r0, rows_per_tile)],
                            out_hbm.at[c, half, pl.ds(r0, rows_per_tile)])

    return spmm_kernel


def kernel(x, edge_index, W, b):
    n, d = x.shape
    c_dim = W.shape[1]
    e = edge_index.shape[1]

    blk = 1024
    n_pad = ((n + blk - 1) // blk) * blk
    ept = -(-e // (NC * NS))
    cpt = -(-ept // CH)
    q = max(8, NB)
    cpt = ((cpt + q - 1) // q) * q
    e_pad = NC * NS * cpt * CH

    src = edge_index[0]
    dst = edge_index[1]
    pad_e = e_pad - e
    src_p = jnp.concatenate(
        [src, jnp.zeros((pad_e,), dtype=src.dtype)]).reshape(-1, CH)
    dst_p = jnp.concatenate(
        [dst, jnp.full((pad_e,), n_pad - 1, dtype=dst.dtype)]).reshape(-1, CH)

    x_p = jnp.zeros((n_pad, d), x.dtype).at[:n].set(x)
    zeros1d = jnp.zeros((n_pad,), jnp.float32)
    ones_ch = jnp.ones((CH,), jnp.float32)

    grid = n_pad // blk

    h0 = pl.pallas_call(
        _pre_body,
        grid=(grid,),
        in_specs=[
            pl.BlockSpec((blk, d), lambda i: (i, 0)),
            pl.BlockSpec((d, c_dim), lambda i: (0, 0)),
            pl.BlockSpec((1, c_dim), lambda i: (0, 0)),
        ],
        out_specs=pl.BlockSpec((blk, c_dim), lambda i: (i, 0)),
        out_shape=jax.ShapeDtypeStruct((n_pad, c_dim), jnp.float32),
    )(x_p, W, b.reshape(1, c_dim))

    deg_kernel = _make_deg_kernel(n_pad, cpt)
    degs = deg_kernel(dst_p, ones_ch, zeros1d)

    nb = n_pad // 128
    degs2 = degs.reshape(2 * nb, 128)
    invb80, dinv80, sq80 = pl.pallas_call(
        _prep_body,
        out_shape=[jax.ShapeDtypeStruct((nb, 128), jnp.float32)] * 3,
    )(degs2)

    dinv_b = jnp.broadcast_to(dinv80.reshape(n_pad, 1), (n_pad, c_dim))
    invb16 = jnp.broadcast_to(invb80.reshape(n_pad, 1), (n_pad, 16))
    sq_b = jnp.broadcast_to(sq80.reshape(n_pad, 1), (n_pad, c_dim))

    u, u0a = pl.pallas_call(
        _scale_body,
        grid=(grid,),
        in_specs=[pl.BlockSpec((blk, c_dim), lambda i: (i, 0))] * 2,
        out_specs=[pl.BlockSpec((blk, c_dim), lambda i: (i, 0))] * 2,
        out_shape=[jax.ShapeDtypeStruct((n_pad, c_dim), jnp.float32)] * 2,
    )(h0, dinv_b)

    fused_kernel = _make_fused_kernel(n_pad, c_dim, cpt)
    u_out = fused_kernel(u0a, invb16, src_p, dst_p)
    u = jnp.concatenate([u_out[0], u_out[1]], axis=1)

    out = pl.pallas_call(
        _post_body,
        grid=(grid,),
        in_specs=[pl.BlockSpec((blk, c_dim), lambda i: (i, 0))] * 2,
        out_specs=pl.BlockSpec((blk, c_dim), lambda i: (i, 0)),
        out_shape=jax.ShapeDtypeStruct((n_pad, c_dim), jnp.float32),
    )(u, sq_b)

    return out[:n]


# speedup vs baseline: 27.9415x; 1.0048x over previous
import functools
import jax
import jax.numpy as jnp
from jax import lax
from jax.experimental import pallas as pl
from jax.experimental.pallas import tpu as pltpu
from jax.experimental.pallas import tpu_sc as plsc

ALPHA = 0.1
KSTEPS = 10

NC = 2
NS = 16
CH = 128
NB = 4


def _pre_body(x_ref, w_ref, b_ref, o_ref):
    h = jnp.dot(x_ref[...], w_ref[...], preferred_element_type=jnp.float32)
    o_ref[...] = jnp.maximum(h + b_ref[...], 0.0)


def _prep_body(d_ref, invb_ref, dinv_ref, sq_ref):
    nb = d_ref.shape[0] // 2
    deg = d_ref[:nb] + d_ref[nb:] + 1.0
    dinv = lax.rsqrt(deg)
    invb_ref[...] = (1.0 - ALPHA) / deg
    dinv_ref[...] = dinv
    sq_ref[...] = deg * dinv


def _scale_body(h_ref, dinv_ref, u_ref, u0a_ref):
    u = h_ref[...] * dinv_ref[...]
    u_ref[...] = u
    u0a_ref[...] = ALPHA * u


def _post_body(u_ref, sq_ref, o_ref):
    z = u_ref[...] * sq_ref[...]
    m = jnp.max(z, axis=1, keepdims=True)
    e = jnp.exp(z - m)
    s = jnp.sum(e, axis=1, keepdims=True)
    o_ref[...] = (z - m) - jnp.log(s)


def _make_deg_kernel(n_pad, cpt):
    mesh = plsc.VectorSubcoreMesh(core_axis_name="c", subcore_axis_name="s", num_cores=NC, num_subcores=NS)
    rows_per_tile = n_pad // NS

    @functools.partial(
        pl.kernel,
        out_type=jax.ShapeDtypeStruct((NC, n_pad), jnp.float32),
        mesh=mesh,
        scratch_types=[
            pltpu.VMEM((cpt, CH), jnp.int32),
            pltpu.VMEM((CH,), jnp.float32),
            pltpu.VMEM_SHARED((n_pad,), jnp.float32),
        ],
    )
    def deg_kernel(dst_hbm, ones_hbm, zeros_hbm, out_hbm, didx, ones_v, acc):
        c = lax.axis_index("c")
        s = lax.axis_index("s")
        wid = c * NS + s
        pltpu.sync_copy(dst_hbm.at[pl.ds(wid * cpt, cpt)], didx)
        pltpu.sync_copy(ones_hbm, ones_v)
        r0 = s * rows_per_tile
        pltpu.sync_copy(zeros_hbm.at[pl.ds(r0, rows_per_tile)],
                        acc.at[pl.ds(r0, rows_per_tile)])
        plsc.subcore_barrier()

        def chunk(j, carry):
            pltpu.sync_copy(ones_v, acc.at[didx.at[j]], add=True)
            return carry

        lax.fori_loop(0, cpt, chunk, 0)
        plsc.subcore_barrier()
        pltpu.sync_copy(acc.at[pl.ds(r0, rows_per_tile)],
                        out_hbm.at[c, pl.ds(r0, rows_per_tile)])

    return deg_kernel


def _make_fused_kernel(n_pad, c_dim, cpt):
    mesh = plsc.VectorSubcoreMesh(core_axis_name="c", subcore_axis_name="s",
                                  num_cores=NC, num_subcores=NS)
    rpt = n_pad // NS
    ch = c_dim // NC
    zr = rpt // 4

    @functools.partial(
        pl.kernel,
        out_type=jax.ShapeDtypeStruct((NC, n_pad, ch), jnp.float32),
        mesh=mesh,
        scratch_types=[
            pltpu.VMEM((2, NB, CH), jnp.int32),
            pltpu.VMEM((2, NB, CH), jnp.int32),
            pltpu.VMEM((NB, CH, ch), jnp.float32),
            pltpu.VMEM((zr, ch), jnp.float32),
            pltpu.VMEM((rpt // 2, ch), jnp.float32),
            pltpu.VMEM((rpt, ch), jnp.float32),
            pltpu.VMEM((rpt, 16), jnp.float32),
            pltpu.VMEM((rpt, ch), jnp.float32),
            pltpu.SemaphoreType.DMA,
            pltpu.SemaphoreType.DMA,
            pltpu.SemaphoreType.DMA,
            pltpu.SemaphoreType.DMA,
            pltpu.VMEM_SHARED((n_pad, ch), jnp.float32),
            pltpu.VMEM_SHARED((n_pad, ch), jnp.float32),
        ],
        compiler_params=pltpu.CompilerParams(use_tc_tiling_on_sc=False),
    )
    def fused_kernel(u0a_hbm, invb16_hbm, src_hbm, dst_hbm,
                     u_out,
                     sidx, didx, rowbuf, zbuf, pbuf, ubuf, invb16,
                     u0ar, gsem, ssem, isem, jsem, agg, ush):
        c = lax.axis_index("c")
        s = lax.axis_index("s")
        r0 = s * rpt
        ngroups = cpt * 2 // NB
        gbase = s * (cpt * 2)

        pltpu.sync_copy(invb16_hbm.at[pl.ds(r0, rpt)], invb16)
        pltpu.sync_copy(
            u0a_hbm.at[pl.ds(r0, rpt), pl.ds(c * ch, ch)], u0ar)
        inv_alpha = jnp.full((16,), 1.0 / ALPHA, jnp.float32)

        def u0row(i, carry):
            for kq in range(ch // 16):
                ubuf[i, pl.ds(kq * 16, 16)] = (
                    u0ar[i, pl.ds(kq * 16, 16)] * inv_alpha)
            return carry

        lax.fori_loop(0, rpt, u0row, 0)
        pltpu.sync_copy(ubuf, ush.at[pl.ds(r0, rpt)])

        z16 = jnp.zeros((16,), jnp.float32)

        def zrow(i, carry):
            for k in range(ch // 16):
                zbuf[i, pl.ds(k * 16, 16)] = z16
            return carry

        lax.fori_loop(0, zr, zrow, 0)
        for q in range(4):
            pltpu.sync_copy(zbuf, agg.at[pl.ds(r0 + q * zr, zr)])
        plsc.subcore_barrier()

        def step(_k, carry):
            pltpu.async_copy(src_hbm.at[pl.ds(gbase, NB)], sidx.at[0], isem)
            pltpu.async_copy(dst_hbm.at[pl.ds(gbase, NB)], didx.at[0], isem)

            def process(slot):
                gd = [pltpu.async_copy(ush.at[sidx.at[slot, b]],
                                       rowbuf.at[b], gsem)
                      for b in range(NB)]
                sd = []
                for b in range(NB):
                    gd[b].wait()
                    sd.append(pltpu.async_copy(rowbuf.at[b],
                                               agg.at[didx.at[slot, b]],
                                               ssem, add=True))
                for b in range(NB):
                    sd[b].wait()

            def drain(slot, sem):
                pltpu.make_async_copy(src_hbm.at[pl.ds(gbase, NB)],
                                      sidx.at[slot], sem).wait()
                pltpu.make_async_copy(dst_hbm.at[pl.ds(gbase, NB)],
                                      didx.at[slot], sem).wait()

            def pair(h, carry2):
                g1 = h * 2 + 1
                pltpu.async_copy(src_hbm.at[pl.ds(gbase + g1 * NB, NB)],
                                 sidx.at[1], jsem)
                pltpu.async_copy(dst_hbm.at[pl.ds(gbase + g1 * NB, NB)],
                                 didx.at[1], jsem)
                drain(0, isem)
                process(0)

                @pl.when(h + 1 < ngroups // 2)
                def _():
                    g2 = h * 2 + 2
                    pltpu.async_copy(src_hbm.at[pl.ds(gbase + g2 * NB, NB)],
                                     sidx.at[0], isem)
                    pltpu.async_copy(dst_hbm.at[pl.ds(gbase + g2 * NB, NB)],
                                     didx.at[0], isem)

                drain(1, jsem)
                process(1)
                return carry2

            lax.fori_loop(0, ngroups // 2, pair, 0)
            plsc.subcore_barrier()
            hrpt = rpt // 2
            for hp in range(2):
                pltpu.sync_copy(agg.at[pl.ds(r0 + hp * hrpt, hrpt)], pbuf)
                for q in range(2):
                    pltpu.sync_copy(
                        zbuf, agg.at[pl.ds(r0 + hp * hrpt + q * zr, zr)])

                def urow(i, carry3):
                    ii = hp * hrpt + i
                    iv = invb16[ii, pl.ds(0, 16)]
                    for kk in range(ch // 16):
                        v = (pbuf[i, pl.ds(kk * 16, 16)]
                             + ubuf[ii, pl.ds(kk * 16, 16)])
                        ubuf[ii, pl.ds(kk * 16, 16)] = (
                            iv * v + u0ar[ii, pl.ds(kk * 16, 16)])
                    return carry3

                lax.fori_loop(0, hrpt, urow, 0)
            pltpu.sync_copy(ubuf, ush.at[pl.ds(r0, rpt)])
            plsc.subcore_barrier()
            return carry

        lax.fori_loop(0, KSTEPS, step, 0)
        pltpu.sync_copy(ubuf, u_out.at[c, pl.ds(r0, rpt)])

    return fused_kernel


def kernel(x, edge_index, W, b):
    n, d = x.shape
    c_dim = W.shape[1]
    e = edge_index.shape[1]

    blk = 1024
    n_pad = ((n + blk - 1) // blk) * blk
    ept = -(-e // (NC * NS))
    cpt = -(-ept // CH)
    q = max(8, NB)
    cpt = ((cpt + q - 1) // q) * q
    e_pad = NC * NS * cpt * CH

    src = edge_index[0]
    dst = edge_index[1]
    pad_e = e_pad - e
    src_p = jnp.concatenate(
        [src, jnp.zeros((pad_e,), dtype=src.dtype)]).reshape(-1, CH)
    dst_p = jnp.concatenate(
        [dst, jnp.full((pad_e,), n_pad - 1, dtype=dst.dtype)]).reshape(-1, CH)

    x_p = jnp.zeros((n_pad, d), x.dtype).at[:n].set(x)
    zeros1d = jnp.zeros((n_pad,), jnp.float32)
    ones_ch = jnp.ones((CH,), jnp.float32)

    grid = n_pad // blk

    h0 = pl.pallas_call(
        _pre_body,
        grid=(grid,),
        in_specs=[
            pl.BlockSpec((blk, d), lambda i: (i, 0)),
            pl.BlockSpec((d, c_dim), lambda i: (0, 0)),
            pl.BlockSpec((1, c_dim), lambda i: (0, 0)),
        ],
        out_specs=pl.BlockSpec((blk, c_dim), lambda i: (i, 0)),
        out_shape=jax.ShapeDtypeStruct((n_pad, c_dim), jnp.float32),
    )(x_p, W, b.reshape(1, c_dim))

    deg_kernel = _make_deg_kernel(n_pad, cpt)
    degs = deg_kernel(dst_p, ones_ch, zeros1d)

    nb = n_pad // 128
    degs2 = degs.reshape(2 * nb, 128)
    invb80, dinv80, sq80 = pl.pallas_call(
        _prep_body,
        out_shape=[jax.ShapeDtypeStruct((nb, 128), jnp.float32)] * 3,
    )(degs2)

    dinv_b = jnp.broadcast_to(dinv80.reshape(n_pad, 1), (n_pad, c_dim))
    invb16 = jnp.broadcast_to(invb80.reshape(n_pad, 1), (n_pad, 16))
    sq_b = jnp.broadcast_to(sq80.reshape(n_pad, 1), (n_pad, c_dim))

    u, u0a = pl.pallas_call(
        _scale_body,
        grid=(grid,),
        in_specs=[pl.BlockSpec((blk, c_dim), lambda i: (i, 0))] * 2,
        out_specs=[pl.BlockSpec((blk, c_dim), lambda i: (i, 0))] * 2,
        out_shape=[jax.ShapeDtypeStruct((n_pad, c_dim), jnp.float32)] * 2,
    )(h0, dinv_b)

    fused_kernel = _make_fused_kernel(n_pad, c_dim, cpt)
    u_out = fused_kernel(u0a, invb16, src_p, dst_p)
    u = jnp.concatenate([u_out[0], u_out[1]], axis=1)

    out = pl.pallas_call(
        _post_body,
        grid=(grid,),
        in_specs=[pl.BlockSpec((blk, c_dim), lambda i: (i, 0))] * 2,
        out_specs=pl.BlockSpec((blk, c_dim), lambda i: (i, 0)),
        out_shape=jax.ShapeDtypeStruct((n_pad, c_dim), jnp.float32),
    )(u, sq_b)

    return out[:n]
